# baseline (speedup 1.0000x reference)
import functools

import jax
import jax.numpy as jnp
from jax import lax
from jax.experimental import pallas as pl
from jax.experimental.pallas import tpu as pltpu
from jax.experimental.pallas import tpu_sc as plsc

N = 2226
E = 6525
R = 1800

NC = 2
NS = 16
NW = NC * NS

PER_R = 64
PER_E = 208
PER_N = 80

S_REFL = PER_R
S_LENE = PER_E + 5
S_EDGE = 2 * PER_E + 2
S_NODE = PER_N + 2


def _align8(x):
    return (x + 7) // 8 * 8


O_N10 = 0
O_ROPE = _align8(O_N10 + 10 * N)
O_LENE = _align8(O_ROPE + N)
O_REFL = _align8(O_LENE + E)
O_EDGE = _align8(O_REFL + R)
O_CONST = _align8(O_EDGE + 2 * E)
F_TOTAL = O_CONST + 16

_F32 = jnp.float32
_I32 = jnp.int32


def _sqrt16(ss):
    i = lax.bitcast_convert_type(ss, _I32)
    y = lax.bitcast_convert_type(
        jnp.int32(0x5F3759DF) - lax.shift_right_logical(i, 1), _F32)
    for _ in range(3):
        y = y * (1.5 - 0.5 * ss * y * y)
    return jnp.where(ss > 0.0, ss * y, 0.0)


def _body(fbuf_h,
          loss_o, c_o, ceq_o, stre_o,
          n10_v, rope_v, refl_v, edge_v, lene_v,
          consts_v,
          loss_s, c_s, ceq_s, stre_s, sem):
    wid = lax.axis_index("s") * NC + lax.axis_index("c")

    base_r = wid * PER_R
    base_e = wid * PER_E
    base_n = wid * PER_N

    b_refl = jnp.minimum(base_r, R - S_REFL)
    b_lene = jnp.minimum(base_e, E - S_LENE)
    b_edge = jnp.minimum(2 * base_e, 2 * E - S_EDGE)
    b_node = jnp.minimum(base_n, N - S_NODE)

    pairs = [
        (fbuf_h.at[pl.ds(O_N10, 10 * N)], n10_v),
        (fbuf_h.at[pl.ds(O_ROPE + b_node, S_NODE)], rope_v),
        (fbuf_h.at[pl.ds(O_LENE + b_lene, S_LENE)], lene_v),
        (fbuf_h.at[pl.ds(O_CONST, 16)], consts_v),
        (fbuf_h.at[pl.ds(O_REFL + b_refl, S_REFL)], refl_v),
        (fbuf_h.at[pl.ds(O_EDGE + b_edge, S_EDGE)], edge_v),
    ]
    handles = [pltpu.async_copy(src, dst, sem) for src, dst in pairs]
    for h in handles:
        h.wait()

    iota = lax.iota(_I32, 16)

    cv = consts_v[...]
    r00, r01, r02 = cv[0], cv[1], cv[2]
    r10, r11, r12 = cv[3], cv[4], cv[5]
    r20, r21, r22 = cv[6], cv[7], cv[8]
    fx, fy, fz = cv[9], cv[10], cv[11]
    bias2 = cv[12] * 2.0 + 440.0

    def _loss_chunk(j, _):
        ii = jnp.minimum(base_r + j * 16 + iota, R - 1)
        ridx = plsc.load_gather(refl_v, [ii - b_refl]).astype(_I32) * 10
        px = plsc.load_gather(n10_v, [ridx])
        py = plsc.load_gather(n10_v, [ridx + 1])
        pz = plsc.load_gather(n10_v, [ridx + 2])
        rx = px * r00 + py * r10 + pz * r20
        ry = px * r01 + py * r11 + pz * r21
        rz = px * r02 + py * r12 + pz * r22
        ex = rx - fx
        ey = ry - fy
        ez = rz - fz
        dis = _sqrt16(ex * ex + ey * ey + ez * ez)
        t = jnp.abs(dis - (rz + bias2)) - 1.0
        loss_s[pl.ds(j * 16, 16)] = jnp.maximum(t, 0.0)
        return 0

    lax.fori_loop(0, PER_R // 16, _loss_chunk, 0, unroll=False)

    def _edge_chunk(j, _):
        ii = jnp.minimum(base_e + j * 16 + iota, E - 1)
        ia = plsc.load_gather(edge_v, [ii * 2 - b_edge]).astype(_I32) * 10
        ib = plsc.load_gather(edge_v, [ii * 2 + 1 - b_edge]).astype(_I32) * 10
        dx = plsc.load_gather(n10_v, [ia]) - plsc.load_gather(n10_v, [ib])
        dy = (plsc.load_gather(n10_v, [ia + 1])
              - plsc.load_gather(n10_v, [ib + 1]))
        dz = (plsc.load_gather(n10_v, [ia + 2])
              - plsc.load_gather(n10_v, [ib + 2]))
        lens = _sqrt16(dx * dx + dy * dy + dz * dz)
        le = plsc.load_gather(lene_v, [ii - b_lene])
        c = jnp.maximum(jnp.abs(lens - le) - 0.007 * le, 0.0) * 100.0
        c_s[pl.ds(j * 16, 16)] = c
        return 0

    lax.fori_loop(0, PER_E // 16, _edge_chunk, 0, unroll=False)

    def _node_chunk(j, _):
        ii = jnp.minimum(base_n + j * 16 + iota, N - 1)
        i10 = ii * 10
        s = plsc.load_gather(n10_v, [i10 + 9])
        rx = (plsc.load_gather(n10_v, [i10 + 3])
              + plsc.load_gather(n10_v, [i10 + 6]) * s
              - plsc.load_gather(n10_v, [i10]))
        ry = (plsc.load_gather(n10_v, [i10 + 4])
              + plsc.load_gather(n10_v, [i10 + 7]) * s
              - plsc.load_gather(n10_v, [i10 + 1]))
        rz = (plsc.load_gather(n10_v, [i10 + 5])
              + plsc.load_gather(n10_v, [i10 + 8]) * s
              - plsc.load_gather(n10_v, [i10 + 2]))
        nn = _sqrt16(rx * rx + ry * ry + rz * rz)
        lr = plsc.load_gather(rope_v, [ii - b_node])
        ceq_s[pl.ds(j * 16, 16)] = jnp.abs(lr - nn) * 100.0
        stre_s[pl.ds(j * 16, 16)] = jnp.maximum(jnp.abs(s) - 0.6, 0.0)
        return 0

    lax.fori_loop(0, PER_N // 16, _node_chunk, 0, unroll=False)

    pltpu.sync_copy(loss_s, loss_o.at[pl.ds(base_r, PER_R)])
    pltpu.sync_copy(c_s, c_o.at[pl.ds(base_e, PER_E)])
    pltpu.sync_copy(ceq_s, ceq_o.at[pl.ds(base_n, PER_N)])
    pltpu.sync_copy(stre_s, stre_o.at[pl.ds(base_n, PER_N)])


_sc_call = functools.partial(
    pl.kernel,
    out_type=[
        jax.ShapeDtypeStruct((NW * PER_R,), _F32),
        jax.ShapeDtypeStruct((NW * PER_E,), _F32),
        jax.ShapeDtypeStruct((NW * PER_N,), _F32),
        jax.ShapeDtypeStruct((NW * PER_N,), _F32),
    ],
    mesh=plsc.VectorSubcoreMesh(core_axis_name="c", subcore_axis_name="s",
                                num_cores=NC, num_subcores=NS),
    compiler_params=pltpu.CompilerParams(needs_layout_passes=False),
    scratch_types=[
        pltpu.VMEM((N * 10,), _F32),
        pltpu.VMEM((S_NODE,), _F32),
        pltpu.VMEM((S_REFL,), _F32),
        pltpu.VMEM((S_EDGE,), _F32),
        pltpu.VMEM((S_LENE,), _F32),
        pltpu.VMEM((16,), _F32),
        pltpu.VMEM((PER_R,), _F32),
        pltpu.VMEM((PER_E,), _F32),
        pltpu.VMEM((PER_N,), _F32),
        pltpu.VMEM((PER_N,), _F32),
        pltpu.SemaphoreType.DMA,
    ],
)(_body)


def _zpad(k):
    return jnp.zeros((k,), _F32)


def _flat2(x, m):
    ar = jnp.arange(x.shape[0] * m, dtype=_I32)
    return x[ar // m, ar % m]


def kernel(pos, stretch, bias, rotm, direction, focus, len_edges, act_up,
           len_rope, refl_idx, all_edges):
    n10 = jnp.concatenate([pos, act_up, direction, stretch], axis=1)
    fbuf = jnp.concatenate([
        n10.reshape(-1), _zpad(O_ROPE - (O_N10 + 10 * N)),
        len_rope, _zpad(O_LENE - (O_ROPE + N)),
        len_edges, _zpad(O_REFL - (O_LENE + E)),
        refl_idx.astype(_F32), _zpad(O_EDGE - (O_REFL + R)),
        all_edges.astype(_F32).reshape(-1), _zpad(O_CONST - (O_EDGE + 2 * E)),
        rotm.reshape(-1), focus, bias, _zpad(3),
    ])
    loss_p, c_p, ceq_p, stre_p = _sc_call(fbuf)
    return jnp.concatenate([loss_p[:R], c_p[:E], ceq_p[:N], stre_p[:N]])


# speedup vs baseline: 1.0790x; 1.0790x over previous
import functools

import jax
import jax.numpy as jnp
from jax import lax
from jax.experimental import pallas as pl
from jax.experimental.pallas import tpu as pltpu
from jax.experimental.pallas import tpu_sc as plsc

N = 2226
E = 6525
R = 1800

NC = 1
NS = 16
NW = NC * NS

PER_R = 128
PER_E = 416
PER_N = 144

S_REFL = PER_R
S_LENE = PER_E + 5
S_EDGE = 2 * PER_E + 2
S_NODE = PER_N + 2


def _align8(x):
    return (x + 7) // 8 * 8


O_N10 = 0
O_ROPE = _align8(O_N10 + 10 * N)
O_LENE = _align8(O_ROPE + N)
O_REFL = _align8(O_LENE + E)
O_EDGE = _align8(O_REFL + R)
O_CONST = _align8(O_EDGE + 2 * E)
F_TOTAL = O_CONST + 16

_F32 = jnp.float32
_I32 = jnp.int32


def _sqrt16(ss):
    i = lax.bitcast_convert_type(ss, _I32)
    y = lax.bitcast_convert_type(
        jnp.int32(0x5F3759DF) - lax.shift_right_logical(i, 1), _F32)
    for _ in range(3):
        y = y * (1.5 - 0.5 * ss * y * y)
    return jnp.where(ss > 0.0, ss * y, 0.0)


def _body(fbuf_h,
          loss_o, c_o, ceq_o, stre_o,
          n10_v, rope_v, refl_v, edge_v, lene_v,
          consts_v,
          loss_s, c_s, ceq_s, stre_s, sem):
    wid = lax.axis_index("s") * NC + lax.axis_index("c")

    base_r = wid * PER_R
    base_e = wid * PER_E
    base_n = wid * PER_N

    b_refl = jnp.minimum(base_r, R - S_REFL)
    b_lene = jnp.minimum(base_e, E - S_LENE)
    b_edge = jnp.minimum(2 * base_e, 2 * E - S_EDGE)
    b_node = jnp.minimum(base_n, N - S_NODE)

    pairs = [
        (fbuf_h.at[pl.ds(O_N10, 10 * N)], n10_v),
        (fbuf_h.at[pl.ds(O_ROPE + b_node, S_NODE)], rope_v),
        (fbuf_h.at[pl.ds(O_LENE + b_lene, S_LENE)], lene_v),
        (fbuf_h.at[pl.ds(O_CONST, 16)], consts_v),
        (fbuf_h.at[pl.ds(O_REFL + b_refl, S_REFL)], refl_v),
        (fbuf_h.at[pl.ds(O_EDGE + b_edge, S_EDGE)], edge_v),
    ]
    handles = [pltpu.async_copy(src, dst, sem) for src, dst in pairs]
    for h in handles:
        h.wait()

    iota = lax.iota(_I32, 16)

    cv = consts_v[...]
    r00, r01, r02 = cv[0], cv[1], cv[2]
    r10, r11, r12 = cv[3], cv[4], cv[5]
    r20, r21, r22 = cv[6], cv[7], cv[8]
    fx, fy, fz = cv[9], cv[10], cv[11]
    bias2 = cv[12] * 2.0 + 440.0

    def _loss_chunk(j, _):
        ii = jnp.minimum(base_r + j * 16 + iota, R - 1)
        ridx = plsc.load_gather(refl_v, [ii - b_refl]).astype(_I32) * 10
        px = plsc.load_gather(n10_v, [ridx])
        py = plsc.load_gather(n10_v, [ridx + 1])
        pz = plsc.load_gather(n10_v, [ridx + 2])
        rx = px * r00 + py * r10 + pz * r20
        ry = px * r01 + py * r11 + pz * r21
        rz = px * r02 + py * r12 + pz * r22
        ex = rx - fx
        ey = ry - fy
        ez = rz - fz
        dis = _sqrt16(ex * ex + ey * ey + ez * ez)
        t = jnp.abs(dis - (rz + bias2)) - 1.0
        loss_s[pl.ds(j * 16, 16)] = jnp.maximum(t, 0.0)
        return 0

    lax.fori_loop(0, PER_R // 16, _loss_chunk, 0, unroll=False)

    def _edge_chunk(j, _):
        ii = jnp.minimum(base_e + j * 16 + iota, E - 1)
        ia = plsc.load_gather(edge_v, [ii * 2 - b_edge]).astype(_I32) * 10
        ib = plsc.load_gather(edge_v, [ii * 2 + 1 - b_edge]).astype(_I32) * 10
        dx = plsc.load_gather(n10_v, [ia]) - plsc.load_gather(n10_v, [ib])
        dy = (plsc.load_gather(n10_v, [ia + 1])
              - plsc.load_gather(n10_v, [ib + 1]))
        dz = (plsc.load_gather(n10_v, [ia + 2])
              - plsc.load_gather(n10_v, [ib + 2]))
        lens = _sqrt16(dx * dx + dy * dy + dz * dz)
        le = plsc.load_gather(lene_v, [ii - b_lene])
        c = jnp.maximum(jnp.abs(lens - le) - 0.007 * le, 0.0) * 100.0
        c_s[pl.ds(j * 16, 16)] = c
        return 0

    lax.fori_loop(0, PER_E // 16, _edge_chunk, 0, unroll=False)

    def _node_chunk(j, _):
        ii = jnp.minimum(base_n + j * 16 + iota, N - 1)
        i10 = ii * 10
        s = plsc.load_gather(n10_v, [i10 + 9])
        rx = (plsc.load_gather(n10_v, [i10 + 3])
              + plsc.load_gather(n10_v, [i10 + 6]) * s
              - plsc.load_gather(n10_v, [i10]))
        ry = (plsc.load_gather(n10_v, [i10 + 4])
              + plsc.load_gather(n10_v, [i10 + 7]) * s
              - plsc.load_gather(n10_v, [i10 + 1]))
        rz = (plsc.load_gather(n10_v, [i10 + 5])
              + plsc.load_gather(n10_v, [i10 + 8]) * s
              - plsc.load_gather(n10_v, [i10 + 2]))
        nn = _sqrt16(rx * rx + ry * ry + rz * rz)
        lr = plsc.load_gather(rope_v, [ii - b_node])
        ceq_s[pl.ds(j * 16, 16)] = jnp.abs(lr - nn) * 100.0
        stre_s[pl.ds(j * 16, 16)] = jnp.maximum(jnp.abs(s) - 0.6, 0.0)
        return 0

    lax.fori_loop(0, PER_N // 16, _node_chunk, 0, unroll=False)

    pltpu.sync_copy(loss_s, loss_o.at[pl.ds(base_r, PER_R)])
    pltpu.sync_copy(c_s, c_o.at[pl.ds(base_e, PER_E)])
    pltpu.sync_copy(ceq_s, ceq_o.at[pl.ds(base_n, PER_N)])
    pltpu.sync_copy(stre_s, stre_o.at[pl.ds(base_n, PER_N)])


_sc_call = functools.partial(
    pl.kernel,
    out_type=[
        jax.ShapeDtypeStruct((NW * PER_R,), _F32),
        jax.ShapeDtypeStruct((NW * PER_E,), _F32),
        jax.ShapeDtypeStruct((NW * PER_N,), _F32),
        jax.ShapeDtypeStruct((NW * PER_N,), _F32),
    ],
    mesh=plsc.VectorSubcoreMesh(core_axis_name="c", subcore_axis_name="s",
                                num_cores=NC, num_subcores=NS),
    compiler_params=pltpu.CompilerParams(needs_layout_passes=False),
    scratch_types=[
        pltpu.VMEM((N * 10,), _F32),
        pltpu.VMEM((S_NODE,), _F32),
        pltpu.VMEM((S_REFL,), _F32),
        pltpu.VMEM((S_EDGE,), _F32),
        pltpu.VMEM((S_LENE,), _F32),
        pltpu.VMEM((16,), _F32),
        pltpu.VMEM((PER_R,), _F32),
        pltpu.VMEM((PER_E,), _F32),
        pltpu.VMEM((PER_N,), _F32),
        pltpu.VMEM((PER_N,), _F32),
        pltpu.SemaphoreType.DMA,
    ],
)(_body)


def _zpad(k):
    return jnp.zeros((k,), _F32)


def _flat2(x, m):
    ar = jnp.arange(x.shape[0] * m, dtype=_I32)
    return x[ar // m, ar % m]


def kernel(pos, stretch, bias, rotm, direction, focus, len_edges, act_up,
           len_rope, refl_idx, all_edges):
    n10 = jnp.concatenate([pos, act_up, direction, stretch], axis=1)
    fbuf = jnp.concatenate([
        n10.reshape(-1), _zpad(O_ROPE - (O_N10 + 10 * N)),
        len_rope, _zpad(O_LENE - (O_ROPE + N)),
        len_edges, _zpad(O_REFL - (O_LENE + E)),
        refl_idx.astype(_F32), _zpad(O_EDGE - (O_REFL + R)),
        all_edges.astype(_F32).reshape(-1), _zpad(O_CONST - (O_EDGE + 2 * E)),
        rotm.reshape(-1), focus, bias, _zpad(3),
    ])
    loss_p, c_p, ceq_p, stre_p = _sc_call(fbuf)
    return jnp.concatenate([loss_p[:R], c_p[:E], ceq_p[:N], stre_p[:N]])


# speedup vs baseline: 1.0854x; 1.0059x over previous
import functools

import jax
import jax.numpy as jnp
from jax import lax
from jax.experimental import pallas as pl
from jax.experimental.pallas import tpu as pltpu
from jax.experimental.pallas import tpu_sc as plsc

N = 2226
E = 6525
R = 1800

NC = 1
NS = 16
NW = NC * NS

PER_R = 128
PER_E = 416
PER_N = 144

S_REFL = PER_R
S_LENE = PER_E + 5
S_EDGE = 2 * PER_E + 2
S_NODE = PER_N + 2
S_N10W = 10 * PER_N + 4


def _align8(x):
    return (x + 7) // 8 * 8


O_N10 = 0
O_ROPE = _align8(O_N10 + 10 * N)
O_LENE = _align8(O_ROPE + N)
O_REFL = _align8(O_LENE + E)
O_EDGE = _align8(O_REFL + R)
O_CONST = _align8(O_EDGE + 2 * E)
F_TOTAL = O_CONST + 16

_F32 = jnp.float32
_I32 = jnp.int32


def _sqrt16(ss):
    i = lax.bitcast_convert_type(ss, _I32)
    y = lax.bitcast_convert_type(
        jnp.int32(0x5F3759DF) - lax.shift_right_logical(i, 1), _F32)
    for _ in range(3):
        y = y * (1.5 - 0.5 * ss * y * y)
    return jnp.where(ss > 0.0, ss * y, 0.0)


def _body(fbuf_h,
          loss_o, c_o, ceq_o, stre_o,
          n10_v, n10w_v, rope_v, refl_v, edge_v, lene_v,
          consts_v,
          loss_s, c_s, ceq_s, stre_s, sem_a, sem_b):
    wid = lax.axis_index("s") * NC + lax.axis_index("c")

    base_r = wid * PER_R
    base_e = wid * PER_E
    base_n = wid * PER_N

    b_refl = jnp.minimum(base_r, R - S_REFL)
    b_lene = jnp.minimum(base_e, E - S_LENE)
    b_edge = jnp.minimum(2 * base_e, 2 * E - S_EDGE)
    b_node = jnp.minimum(base_n, N - S_NODE)
    b_n10w = jnp.minimum(10 * base_n, 10 * N - S_N10W)

    h_a = pltpu.async_copy(fbuf_h.at[pl.ds(O_N10, 10 * N)], n10_v, sem_a)
    pairs = [
        (fbuf_h.at[pl.ds(O_N10 + b_n10w, S_N10W)], n10w_v),
        (fbuf_h.at[pl.ds(O_ROPE + b_node, S_NODE)], rope_v),
        (fbuf_h.at[pl.ds(O_LENE + b_lene, S_LENE)], lene_v),
        (fbuf_h.at[pl.ds(O_CONST, 16)], consts_v),
        (fbuf_h.at[pl.ds(O_REFL + b_refl, S_REFL)], refl_v),
        (fbuf_h.at[pl.ds(O_EDGE + b_edge, S_EDGE)], edge_v),
    ]
    handles = [pltpu.async_copy(src, dst, sem_b) for src, dst in pairs]
    for h in handles:
        h.wait()

    iota = lax.iota(_I32, 16)

    cv = consts_v[...]
    r00, r01, r02 = cv[0], cv[1], cv[2]
    r10, r11, r12 = cv[3], cv[4], cv[5]
    r20, r21, r22 = cv[6], cv[7], cv[8]
    fx, fy, fz = cv[9], cv[10], cv[11]
    bias2 = cv[12] * 2.0 + 440.0

    def _node_chunk(j, _):
        ii = jnp.minimum(base_n + j * 16 + iota, N - 1)
        i10 = ii * 10 - b_n10w
        s = plsc.load_gather(n10w_v, [i10 + 9])
        rx = (plsc.load_gather(n10w_v, [i10 + 3])
              + plsc.load_gather(n10w_v, [i10 + 6]) * s
              - plsc.load_gather(n10w_v, [i10]))
        ry = (plsc.load_gather(n10w_v, [i10 + 4])
              + plsc.load_gather(n10w_v, [i10 + 7]) * s
              - plsc.load_gather(n10w_v, [i10 + 1]))
        rz = (plsc.load_gather(n10w_v, [i10 + 5])
              + plsc.load_gather(n10w_v, [i10 + 8]) * s
              - plsc.load_gather(n10w_v, [i10 + 2]))
        nn = _sqrt16(rx * rx + ry * ry + rz * rz)
        lr = plsc.load_gather(rope_v, [ii - b_node])
        ceq_s[pl.ds(j * 16, 16)] = jnp.abs(lr - nn) * 100.0
        stre_s[pl.ds(j * 16, 16)] = jnp.maximum(jnp.abs(s) - 0.6, 0.0)
        return 0

    lax.fori_loop(0, PER_N // 16, _node_chunk, 0, unroll=False)

    h_a.wait()

    def _loss_chunk(j, _):
        ii = jnp.minimum(base_r + j * 16 + iota, R - 1)
        ridx = plsc.load_gather(refl_v, [ii - b_refl]).astype(_I32) * 10
        px = plsc.load_gather(n10_v, [ridx])
        py = plsc.load_gather(n10_v, [ridx + 1])
        pz = plsc.load_gather(n10_v, [ridx + 2])
        rx = px * r00 + py * r10 + pz * r20
        ry = px * r01 + py * r11 + pz * r21
        rz = px * r02 + py * r12 + pz * r22
        ex = rx - fx
        ey = ry - fy
        ez = rz - fz
        dis = _sqrt16(ex * ex + ey * ey + ez * ez)
        t = jnp.abs(dis - (rz + bias2)) - 1.0
        loss_s[pl.ds(j * 16, 16)] = jnp.maximum(t, 0.0)
        return 0

    lax.fori_loop(0, PER_R // 16, _loss_chunk, 0, unroll=False)

    def _edge_chunk(j, _):
        ii = jnp.minimum(base_e + j * 16 + iota, E - 1)
        ia = plsc.load_gather(edge_v, [ii * 2 - b_edge]).astype(_I32) * 10
        ib = plsc.load_gather(edge_v, [ii * 2 + 1 - b_edge]).astype(_I32) * 10
        dx = plsc.load_gather(n10_v, [ia]) - plsc.load_gather(n10_v, [ib])
        dy = (plsc.load_gather(n10_v, [ia + 1])
              - plsc.load_gather(n10_v, [ib + 1]))
        dz = (plsc.load_gather(n10_v, [ia + 2])
              - plsc.load_gather(n10_v, [ib + 2]))
        lens = _sqrt16(dx * dx + dy * dy + dz * dz)
        le = plsc.load_gather(lene_v, [ii - b_lene])
        c = jnp.maximum(jnp.abs(lens - le) - 0.007 * le, 0.0) * 100.0
        c_s[pl.ds(j * 16, 16)] = c
        return 0

    lax.fori_loop(0, PER_E // 16, _edge_chunk, 0, unroll=False)

    pltpu.sync_copy(loss_s, loss_o.at[pl.ds(base_r, PER_R)])
    pltpu.sync_copy(c_s, c_o.at[pl.ds(base_e, PER_E)])
    pltpu.sync_copy(ceq_s, ceq_o.at[pl.ds(base_n, PER_N)])
    pltpu.sync_copy(stre_s, stre_o.at[pl.ds(base_n, PER_N)])


_sc_call = functools.partial(
    pl.kernel,
    out_type=[
        jax.ShapeDtypeStruct((NW * PER_R,), _F32),
        jax.ShapeDtypeStruct((NW * PER_E,), _F32),
        jax.ShapeDtypeStruct((NW * PER_N,), _F32),
        jax.ShapeDtypeStruct((NW * PER_N,), _F32),
    ],
    mesh=plsc.VectorSubcoreMesh(core_axis_name="c", subcore_axis_name="s",
                                num_cores=NC, num_subcores=NS),
    compiler_params=pltpu.CompilerParams(needs_layout_passes=False),
    scratch_types=[
        pltpu.VMEM((N * 10,), _F32),
        pltpu.VMEM((S_N10W,), _F32),
        pltpu.VMEM((S_NODE,), _F32),
        pltpu.VMEM((S_REFL,), _F32),
        pltpu.VMEM((S_EDGE,), _F32),
        pltpu.VMEM((S_LENE,), _F32),
        pltpu.VMEM((16,), _F32),
        pltpu.VMEM((PER_R,), _F32),
        pltpu.VMEM((PER_E,), _F32),
        pltpu.VMEM((PER_N,), _F32),
        pltpu.VMEM((PER_N,), _F32),
        pltpu.SemaphoreType.DMA,
        pltpu.SemaphoreType.DMA,
    ],
)(_body)


def _zpad(k):
    return jnp.zeros((k,), _F32)


def _flat2(x, m):
    ar = jnp.arange(x.shape[0] * m, dtype=_I32)
    return x[ar // m, ar % m]


def kernel(pos, stretch, bias, rotm, direction, focus, len_edges, act_up,
           len_rope, refl_idx, all_edges):
    n10 = jnp.concatenate([pos, act_up, direction, stretch], axis=1)
    fbuf = jnp.concatenate([
        n10.reshape(-1), _zpad(O_ROPE - (O_N10 + 10 * N)),
        len_rope, _zpad(O_LENE - (O_ROPE + N)),
        len_edges, _zpad(O_REFL - (O_LENE + E)),
        refl_idx.astype(_F32), _zpad(O_EDGE - (O_REFL + R)),
        all_edges.astype(_I32).reshape(-1).astype(_F32),
        _zpad(O_CONST - (O_EDGE + 2 * E)),
        rotm.reshape(-1), focus, bias, _zpad(3),
    ])
    loss_p, c_p, ceq_p, stre_p = _sc_call(fbuf)
    return jnp.concatenate([loss_p[:R], c_p[:E], ceq_p[:N], stre_p[:N]])
